# baseline (speedup 1.0000x reference)
import jax
import jax.numpy as jnp
import numpy as np
from jax import lax
from jax.experimental import pallas as pl
from jax.experimental.pallas import tpu as pltpu
from jax.experimental.pallas import tpu_sc as plsc

N = 262144
F = 64
B = 256
NC = 2
NS = 16
NW = NC * NS
SAMP = N // NW
CHUNK = 256
NCHUNK = SAMP // CHUNK
WPS = F // 4
NFG = 4
GSZ = 16 * B
NHB = 4 * NFG
ROW_UNROLL = 8


def _sc_body(x_hbm, g_hbm, h_hbm, out_hbm, x_buf, g_v, h_v, *rest):
    hbufs = rest[:NHB]
    sems = rest[NHB:]
    c = lax.axis_index("c")
    s = lax.axis_index("s")
    wid = s * NC + c
    base = wid * SAMP
    cw = CHUNK * WPS

    def start_x(ci, b):
        pltpu.make_async_copy(
            x_hbm.at[pl.ds((base + ci * CHUNK) * WPS, cw)],
            x_buf.at[pl.ds(b * cw, cw)],
            sems[b],
        ).start()

    def wait_x(b):
        pltpu.make_async_copy(
            x_hbm.at[pl.ds(base * WPS, cw)],
            x_buf.at[pl.ds(b * cw, cw)],
            sems[b],
        ).wait()

    start_x(0, 0)
    start_x(1, 1)
    pltpu.sync_copy(g_hbm.at[pl.ds(base, SAMP)], g_v)
    pltpu.sync_copy(h_hbm.at[pl.ds(base, SAMP)], h_v)

    zeros = jnp.zeros((16,), jnp.float32)

    def zero_body(i, carry):
        for hb in hbufs:
            hb[pl.ds(i * 16, 16)] = zeros
        return carry

    lax.fori_loop(0, GSZ // 16, zero_body, 0)

    lane_off = lax.iota(jnp.int32, 16) * B
    mask255 = jnp.full((16,), 255, jnp.int32)

    def compute_chunk(ci, b):
        def rows_body(r8, carry):
            rows = []
            for u in range(ROW_UNROLL):
                r = r8 * ROW_UNROLL + u
                gi = ci * CHUNK + r
                gidx = jnp.full((16,), gi, jnp.int32)
                gs = plsc.load_gather(g_v, [gidx])
                hs = plsc.load_gather(h_v, [gidx])
                w = x_buf[pl.ds(b * cw + r * WPS, WPS)]
                idxs = [
                    lane_off + ((w >> (8 * j)) & mask255) for j in range(NFG)
                ]
                rows.append((gs, hs, idxs))
            for u, (gs, hs, idxs) in enumerate(rows):
                par = (u % 2) * 2 * NFG
                for j in range(NFG):
                    plsc.addupdate_scatter(hbufs[par + j], [idxs[j]], gs)
                    plsc.addupdate_scatter(hbufs[par + NFG + j], [idxs[j]], hs)
            return carry

        lax.fori_loop(0, CHUNK // ROW_UNROLL, rows_body, 0)

    def step_body(si, carry):
        for b in range(2):
            ci = si * 2 + b
            wait_x(b)
            compute_chunk(ci, b)

            @pl.when(ci + 2 < NCHUNK)
            def _():
                start_x(ci + 2, b)

        return carry

    lax.fori_loop(0, NCHUNK // 2, step_body, 0)

    for k, hb in enumerate(hbufs):
        pltpu.sync_copy(hb, out_hbm.at[wid, k])


_sc_hist = pl.kernel(
    _sc_body,
    out_type=jax.ShapeDtypeStruct((NW, NHB, GSZ), jnp.float32),
    mesh=plsc.VectorSubcoreMesh(
        core_axis_name="c", subcore_axis_name="s", num_cores=NC, num_subcores=NS
    ),
    compiler_params=pltpu.CompilerParams(needs_layout_passes=False),
    scratch_types=[
        pltpu.VMEM((2 * CHUNK * WPS,), jnp.int32),
        pltpu.VMEM((SAMP,), jnp.float32),
        pltpu.VMEM((SAMP,), jnp.float32),
    ]
    + [pltpu.VMEM((GSZ,), jnp.float32) for _ in range(NHB)]
    + [
        pltpu.SemaphoreType.DMA,
        pltpu.SemaphoreType.DMA,
    ],
)


def _tc_body(p_ref, gl_ref, hl_ref):
    acc = jnp.sum(p_ref[...], axis=0)
    bi = lax.broadcasted_iota(jnp.int32, (B, B), 0)
    ki = lax.broadcasted_iota(jnp.int32, (B, B), 1)
    m = (bi > ki).astype(jnp.float32)
    gl_ref[...] = lax.dot(acc[:F], m, precision=lax.Precision.HIGHEST)
    hl_ref[...] = lax.dot(acc[F:], m, precision=lax.Precision.HIGHEST)


_tc_finish = pl.pallas_call(
    _tc_body,
    out_shape=(
        jax.ShapeDtypeStruct((F, B), jnp.float32),
        jax.ShapeDtypeStruct((F, B), jnp.float32),
    ),
)

_PERM = np.array([(f % 4) * 16 + f // 4 for f in range(F)], np.int32)


@jax.jit
def kernel(X, gradient, hessian):
    xp = lax.bitcast_convert_type(
        X.astype(jnp.int8).reshape(N, WPS, 4), jnp.int32
    ).reshape(N * WPS)
    partials = _sc_hist(xp, gradient, hessian)
    gl, hl = _tc_finish(partials.reshape(2 * NW, 2 * F, B))
    return (gl[_PERM][None], hl[_PERM][None])


# speedup vs baseline: 1.4496x; 1.4496x over previous
import jax
import jax.numpy as jnp
import numpy as np
from jax import lax
from jax.experimental import pallas as pl
from jax.experimental.pallas import tpu as pltpu
from jax.experimental.pallas import tpu_sc as plsc

N = 262144
F = 64
B = 256
NC = 2
NS = 16
NW = NC * NS
SAMP = N // NW
CHUNK = 256
NCHUNK = SAMP // CHUNK
WPS = F // 4
NFG = 4
GSZ = 16 * B
NHB = 4 * NFG
ROW_UNROLL = 8


def _sc_body(x_hbm, g_hbm, h_hbm, out_hbm, x_buf, g_v, h_v, *rest):
    hbufs = rest[:NHB]
    sems = rest[NHB:]
    c = lax.axis_index("c")
    s = lax.axis_index("s")
    wid = s * NC + c
    base = wid * SAMP
    cw = CHUNK * WPS

    def start_x(ci, b):
        pltpu.make_async_copy(
            x_hbm.at[pl.ds((base + ci * CHUNK) * WPS, cw)],
            x_buf.at[pl.ds(b * cw, cw)],
            sems[b],
        ).start()

    def wait_x(b):
        pltpu.make_async_copy(
            x_hbm.at[pl.ds(base * WPS, cw)],
            x_buf.at[pl.ds(b * cw, cw)],
            sems[b],
        ).wait()

    start_x(0, 0)
    start_x(1, 1)
    pltpu.sync_copy(g_hbm.at[pl.ds(base, SAMP)], g_v)
    pltpu.sync_copy(h_hbm.at[pl.ds(base, SAMP)], h_v)

    zeros = jnp.zeros((16,), jnp.float32)

    def zero_body(i, carry):
        for hb in hbufs:
            hb[pl.ds(i * 16, 16)] = zeros
        return carry

    lax.fori_loop(0, GSZ // 16, zero_body, 0)

    lane_off = lax.iota(jnp.int32, 16) * B
    mask255 = jnp.full((16,), 255, jnp.int32)

    def compute_chunk(ci, b):
        def rows_body(r8, carry):
            rows = []
            for u in range(ROW_UNROLL):
                r = r8 * ROW_UNROLL + u
                gi = ci * CHUNK + r
                gidx = jnp.full((16,), gi, jnp.int32)
                gs = plsc.load_gather(g_v, [gidx])
                hs = plsc.load_gather(h_v, [gidx])
                w = x_buf[pl.ds(b * cw + r * WPS, WPS)]
                idxs = [
                    lane_off + ((w >> (8 * j)) & mask255) for j in range(NFG)
                ]
                rows.append((gs, hs, idxs))
            for u, (gs, hs, idxs) in enumerate(rows):
                par = (u % 2) * 2 * NFG
                for j in range(NFG):
                    plsc.addupdate_scatter(hbufs[par + j], [idxs[j]], gs)
                    plsc.addupdate_scatter(hbufs[par + NFG + j], [idxs[j]], hs)
            return carry

        lax.fori_loop(0, CHUNK // ROW_UNROLL, rows_body, 0)

    def step_body(si, carry):
        for b in range(2):
            ci = si * 2 + b
            wait_x(b)
            compute_chunk(ci, b)

            @pl.when(ci + 2 < NCHUNK)
            def _():
                start_x(ci + 2, b)

        return carry

    lax.fori_loop(0, NCHUNK // 2, step_body, 0)

    for k, hb in enumerate(hbufs):
        pltpu.sync_copy(hb, out_hbm.at[wid, k])


_sc_hist = pl.kernel(
    _sc_body,
    out_type=jax.ShapeDtypeStruct((NW, NHB, GSZ), jnp.float32),
    mesh=plsc.VectorSubcoreMesh(
        core_axis_name="c", subcore_axis_name="s", num_cores=NC, num_subcores=NS
    ),
    compiler_params=pltpu.CompilerParams(needs_layout_passes=False),
    scratch_types=[
        pltpu.VMEM((2 * CHUNK * WPS,), jnp.int32),
        pltpu.VMEM((SAMP,), jnp.float32),
        pltpu.VMEM((SAMP,), jnp.float32),
    ]
    + [pltpu.VMEM((GSZ,), jnp.float32) for _ in range(NHB)]
    + [
        pltpu.SemaphoreType.DMA,
        pltpu.SemaphoreType.DMA,
    ],
)


BS = 2048


def _tc_pack_body(x_ref, o_ref):
    fi = lax.broadcasted_iota(jnp.int32, (F, 128), 0)
    ci = lax.broadcasted_iota(jnp.int32, (F, 128), 1)
    sel = (fi >> 2) == (ci & 15)
    byte = fi & 3
    w0 = jnp.where(sel & (byte == 0), 1.0, 0.0)
    w1 = jnp.where(sel & (byte == 1), 256.0, 0.0)
    plo = (w0 + w1).astype(jnp.bfloat16)
    w2 = jnp.where(sel & (byte == 2), 1.0, 0.0)
    w3 = jnp.where(sel & (byte == 3), 256.0, 0.0)
    phi = (w2 + w3).astype(jnp.bfloat16)
    x = x_ref[...].astype(jnp.bfloat16)
    dn = (((0,), (0,)), ((), ()))
    lo = lax.dot_general(x, plo, dn, preferred_element_type=jnp.float32)
    hi = lax.dot_general(x, phi, dn, preferred_element_type=jnp.float32)
    sub = lax.broadcasted_iota(jnp.int32, (BS, 128), 0) & 7
    cg = lax.broadcasted_iota(jnp.int32, (BS, 128), 1) >> 4
    msk = sub == cg
    lo8 = jnp.where(msk, lo, 0.0).reshape(BS // 8, 8, 128).sum(axis=1)
    hi8 = jnp.where(msk, hi, 0.0).reshape(BS // 8, 8, 128).sum(axis=1)
    o_ref[...] = lo8.astype(jnp.int32) | (hi8.astype(jnp.int32) << 16)


_tc_pack = pl.pallas_call(
    _tc_pack_body,
    grid=(N // BS,),
    in_specs=[pl.BlockSpec((F, BS), lambda i: (0, i))],
    out_specs=pl.BlockSpec((BS // 8, 128), lambda i: (i, 0)),
    out_shape=jax.ShapeDtypeStruct((N // 8, 128), jnp.int32),
)


def _tc_body(p_ref, gl_ref, hl_ref):
    acc = jnp.sum(p_ref[...], axis=0)
    bi = lax.broadcasted_iota(jnp.int32, (B, B), 0)
    ki = lax.broadcasted_iota(jnp.int32, (B, B), 1)
    m = (bi > ki).astype(jnp.float32)
    gl_ref[...] = lax.dot(acc[:F], m, precision=lax.Precision.HIGHEST)
    hl_ref[...] = lax.dot(acc[F:], m, precision=lax.Precision.HIGHEST)


_tc_finish = pl.pallas_call(
    _tc_body,
    out_shape=(
        jax.ShapeDtypeStruct((F, B), jnp.float32),
        jax.ShapeDtypeStruct((F, B), jnp.float32),
    ),
)

_PERM = np.array([(f % 4) * 16 + f // 4 for f in range(F)], np.int32)


@jax.jit
def kernel(X, gradient, hessian):
    xp = _tc_pack(X.T).reshape(N * WPS)
    partials = _sc_hist(xp, gradient, hessian)
    gl, hl = _tc_finish(partials.reshape(2 * NW, 2 * F, B))
    return (gl[_PERM][None], hl[_PERM][None])


# speedup vs baseline: 1.4503x; 1.0005x over previous
import jax
import jax.numpy as jnp
import numpy as np
from jax import lax
from jax.experimental import pallas as pl
from jax.experimental.pallas import tpu as pltpu
from jax.experimental.pallas import tpu_sc as plsc

N = 262144
F = 64
B = 256
NC = 2
NS = 16
NW = NC * NS
SAMP = N // NW
CHUNK = 256
NCHUNK = SAMP // CHUNK
WPS = F // 4
NFG = 4
GSZ = 16 * B
NHB = 4 * NFG
ROW_UNROLL = 8


def _sc_body(x_hbm, g_hbm, h_hbm, out_hbm, x_buf, g_v, h_v, *rest):
    hbufs = rest[:NHB]
    sems = rest[NHB:]
    c = lax.axis_index("c")
    s = lax.axis_index("s")
    wid = s * NC + c
    base = wid * SAMP
    cw = CHUNK * WPS

    def start_x(ci, b):
        pltpu.make_async_copy(
            x_hbm.at[pl.ds((base + ci * CHUNK) * WPS, cw)],
            x_buf.at[pl.ds(b * cw, cw)],
            sems[b],
        ).start()

    def wait_x(b):
        pltpu.make_async_copy(
            x_hbm.at[pl.ds(base * WPS, cw)],
            x_buf.at[pl.ds(b * cw, cw)],
            sems[b],
        ).wait()

    start_x(0, 0)
    start_x(1, 1)
    pltpu.sync_copy(g_hbm.at[pl.ds(base, SAMP)], g_v)
    pltpu.sync_copy(h_hbm.at[pl.ds(base, SAMP)], h_v)

    zeros = jnp.zeros((16,), jnp.float32)

    def zero_body(i, carry):
        for hb in hbufs:
            hb[pl.ds(i * 16, 16)] = zeros
        return carry

    lax.fori_loop(0, GSZ // 16, zero_body, 0)

    lane_off = lax.iota(jnp.int32, 16) * B
    mask255 = jnp.full((16,), 255, jnp.int32)

    def compute_chunk(ci, b):
        def rows_body(r8, carry):
            rows = []
            for u in range(ROW_UNROLL):
                r = r8 * ROW_UNROLL + u
                gi = ci * CHUNK + r
                gidx = jnp.full((16,), gi, jnp.int32)
                gs = plsc.load_gather(g_v, [gidx])
                hs = plsc.load_gather(h_v, [gidx])
                w = x_buf[pl.ds(b * cw + r * WPS, WPS)]
                idxs = [
                    lane_off + ((w >> (8 * j)) & mask255) for j in range(NFG)
                ]
                rows.append((gs, hs, idxs))
            for u, (gs, hs, idxs) in enumerate(rows):
                par = (u % 2) * 2 * NFG
                for j in range(NFG):
                    plsc.addupdate_scatter(hbufs[par + j], [idxs[j]], gs)
                    plsc.addupdate_scatter(hbufs[par + NFG + j], [idxs[j]], hs)
            return carry

        lax.fori_loop(0, CHUNK // ROW_UNROLL, rows_body, 0)

    def step_body(si, carry):
        for b in range(2):
            ci = si * 2 + b
            wait_x(b)
            compute_chunk(ci, b)

            @pl.when(ci + 2 < NCHUNK)
            def _():
                start_x(ci + 2, b)

        return carry

    lax.fori_loop(0, NCHUNK // 2, step_body, 0)

    for k, hb in enumerate(hbufs):
        pltpu.sync_copy(hb, out_hbm.at[wid, k])


_sc_hist = pl.kernel(
    _sc_body,
    out_type=jax.ShapeDtypeStruct((NW, NHB, GSZ), jnp.float32),
    mesh=plsc.VectorSubcoreMesh(
        core_axis_name="c", subcore_axis_name="s", num_cores=NC, num_subcores=NS
    ),
    compiler_params=pltpu.CompilerParams(needs_layout_passes=False),
    scratch_types=[
        pltpu.VMEM((2 * CHUNK * WPS,), jnp.int32),
        pltpu.VMEM((SAMP,), jnp.float32),
        pltpu.VMEM((SAMP,), jnp.float32),
    ]
    + [pltpu.VMEM((GSZ,), jnp.float32) for _ in range(NHB)]
    + [
        pltpu.SemaphoreType.DMA,
        pltpu.SemaphoreType.DMA,
    ],
)


BS = 2048


def _tc_pack_body(x_ref, o_ref):
    fi = lax.broadcasted_iota(jnp.int32, (F, 128), 0)
    ci = lax.broadcasted_iota(jnp.int32, (F, 128), 1)
    sel = (fi >> 2) == (ci & 15)
    byte = fi & 3
    w0 = jnp.where(sel & (byte == 0), 1.0, 0.0)
    w1 = jnp.where(sel & (byte == 1), 256.0, 0.0)
    plo = (w0 + w1).astype(jnp.bfloat16)
    w2 = jnp.where(sel & (byte == 2), 1.0, 0.0)
    w3 = jnp.where(sel & (byte == 3), 256.0, 0.0)
    phi = (w2 + w3).astype(jnp.bfloat16)
    x = x_ref[...].astype(jnp.bfloat16)
    dn = (((0,), (0,)), ((), ()))
    lo = lax.dot_general(x, plo, dn, precision=lax.Precision.DEFAULT,
                         preferred_element_type=jnp.float32)
    hi = lax.dot_general(x, phi, dn, precision=lax.Precision.DEFAULT,
                         preferred_element_type=jnp.float32)
    sub = lax.broadcasted_iota(jnp.int32, (BS, 128), 0) & 7
    cg = lax.broadcasted_iota(jnp.int32, (BS, 128), 1) >> 4
    msk = sub == cg
    lo8 = jnp.where(msk, lo, 0.0).reshape(BS // 8, 8, 128).sum(axis=1)
    hi8 = jnp.where(msk, hi, 0.0).reshape(BS // 8, 8, 128).sum(axis=1)
    o_ref[...] = lo8.astype(jnp.int32) | (hi8.astype(jnp.int32) << 16)


_tc_pack = pl.pallas_call(
    _tc_pack_body,
    grid=(N // BS,),
    in_specs=[pl.BlockSpec((F, BS), lambda i: (0, i))],
    out_specs=pl.BlockSpec((BS // 8, 128), lambda i: (i, 0)),
    out_shape=jax.ShapeDtypeStruct((N // 8, 128), jnp.int32),
    compiler_params=pltpu.CompilerParams(fuse_transposed_lhs_in_matmul=True),
)


def _tc_body(p_ref, gl_ref, hl_ref):
    acc = jnp.sum(p_ref[...], axis=0)
    bi = lax.broadcasted_iota(jnp.int32, (B, B), 0)
    ki = lax.broadcasted_iota(jnp.int32, (B, B), 1)
    m = (bi > ki).astype(jnp.float32)
    gl_ref[...] = lax.dot(acc[:F], m, precision=lax.Precision.HIGHEST)
    hl_ref[...] = lax.dot(acc[F:], m, precision=lax.Precision.HIGHEST)


_tc_finish = pl.pallas_call(
    _tc_body,
    out_shape=(
        jax.ShapeDtypeStruct((F, B), jnp.float32),
        jax.ShapeDtypeStruct((F, B), jnp.float32),
    ),
)

_PERM = np.array([(f % 4) * 16 + f // 4 for f in range(F)], np.int32)


@jax.jit
def kernel(X, gradient, hessian):
    xp = _tc_pack(X.T).reshape(N * WPS)
    partials = _sc_hist(xp, gradient, hessian)
    gl, hl = _tc_finish(partials.reshape(2 * NW, 2 * F, B))
    return (gl[_PERM][None], hl[_PERM][None])


# speedup vs baseline: 1.5004x; 1.0345x over previous
import jax
import jax.numpy as jnp
import numpy as np
from jax import lax
from jax.experimental import pallas as pl
from jax.experimental.pallas import tpu as pltpu
from jax.experimental.pallas import tpu_sc as plsc

N = 262144
F = 64
B = 256
NC = 2
NS = 16
NW = NC * NS
SAMP = N // NW
CHUNK = 128
NCHUNK = SAMP // CHUNK
NFG = F // 16
GSZ = 16 * B
NHB = 4 * NFG
ROW_UNROLL = 8
BS = 2048


def _tc_tr_body(x_ref, o_ref):
    fi = lax.broadcasted_iota(jnp.int32, (F, F), 0)
    ci = lax.broadcasted_iota(jnp.int32, (F, F), 1)
    ident = (fi == ci).astype(jnp.bfloat16)
    x = x_ref[...].astype(jnp.bfloat16)
    xt = lax.dot_general(
        x, ident, (((0,), (0,)), ((), ())),
        precision=lax.Precision.DEFAULT,
        preferred_element_type=jnp.float32,
    )
    o_ref[...] = xt.astype(jnp.int32)


_tc_tr = pl.pallas_call(
    _tc_tr_body,
    grid=(N // BS,),
    in_specs=[pl.BlockSpec((F, BS), lambda i: (0, i))],
    out_specs=pl.BlockSpec((BS, F), lambda i: (i, 0)),
    out_shape=jax.ShapeDtypeStruct((N, F), jnp.int32),
    compiler_params=pltpu.CompilerParams(fuse_transposed_lhs_in_matmul=True),
)


def _sc_body(x_hbm, g_hbm, h_hbm, out_hbm, x_buf, g_v, h_v, *rest):
    hbufs = rest[:NHB]
    sems = rest[NHB:]
    c = lax.axis_index("c")
    s = lax.axis_index("s")
    wid = s * NC + c
    base = wid * SAMP

    def start_x(ci, b):
        pltpu.make_async_copy(
            x_hbm.at[pl.ds(base + ci * CHUNK, CHUNK)], x_buf.at[b], sems[b]
        ).start()

    def wait_x(b):
        pltpu.make_async_copy(
            x_hbm.at[pl.ds(base, CHUNK)], x_buf.at[b], sems[b]
        ).wait()

    start_x(0, 0)
    start_x(1, 1)
    pltpu.sync_copy(g_hbm.at[pl.ds(base, SAMP)], g_v)
    pltpu.sync_copy(h_hbm.at[pl.ds(base, SAMP)], h_v)

    zeros = jnp.zeros((16,), jnp.float32)

    def zero_body(i, carry):
        for hb in hbufs:
            hb[pl.ds(i * 16, 16)] = zeros
        return carry

    lax.fori_loop(0, GSZ // 16, zero_body, 0)

    lane_off = lax.iota(jnp.int32, 16) * B

    def compute_chunk(ci, b):
        def rows_body(r8, carry):
            rows = []
            for u in range(ROW_UNROLL):
                r = r8 * ROW_UNROLL + u
                gi = ci * CHUNK + r
                gidx = jnp.full((16,), gi, jnp.int32)
                gs = plsc.load_gather(g_v, [gidx])
                hs = plsc.load_gather(h_v, [gidx])
                idxs = [
                    x_buf[b, r, pl.ds(fg * 16, 16)] + lane_off
                    for fg in range(NFG)
                ]
                rows.append((gs, hs, idxs))
            for u, (gs, hs, idxs) in enumerate(rows):
                par = (u % 2) * 2 * NFG
                for fg in range(NFG):
                    plsc.addupdate_scatter(hbufs[par + fg], [idxs[fg]], gs)
                    plsc.addupdate_scatter(hbufs[par + NFG + fg], [idxs[fg]], hs)
            return carry

        lax.fori_loop(0, CHUNK // ROW_UNROLL, rows_body, 0)

    def step_body(si, carry):
        for b in range(2):
            ci = si * 2 + b
            wait_x(b)
            compute_chunk(ci, b)

            @pl.when(ci + 2 < NCHUNK)
            def _():
                start_x(ci + 2, b)

        return carry

    lax.fori_loop(0, NCHUNK // 2, step_body, 0)

    for k, hb in enumerate(hbufs):
        pltpu.sync_copy(hb, out_hbm.at[wid, k])


_sc_hist = pl.kernel(
    _sc_body,
    out_type=jax.ShapeDtypeStruct((NW, NHB, GSZ), jnp.float32),
    mesh=plsc.VectorSubcoreMesh(
        core_axis_name="c", subcore_axis_name="s", num_cores=NC, num_subcores=NS
    ),
    compiler_params=pltpu.CompilerParams(needs_layout_passes=False),
    scratch_types=[
        pltpu.VMEM((2, CHUNK, F), jnp.int32),
        pltpu.VMEM((SAMP,), jnp.float32),
        pltpu.VMEM((SAMP,), jnp.float32),
    ]
    + [pltpu.VMEM((GSZ,), jnp.float32) for _ in range(NHB)]
    + [
        pltpu.SemaphoreType.DMA,
        pltpu.SemaphoreType.DMA,
    ],
)


def _tc_body(p_ref, gl_ref, hl_ref):
    acc = jnp.sum(p_ref[...], axis=0)
    bi = lax.broadcasted_iota(jnp.int32, (B, B), 0)
    ki = lax.broadcasted_iota(jnp.int32, (B, B), 1)
    m = (bi > ki).astype(jnp.float32)
    gl_ref[...] = lax.dot(acc[:F], m, precision=lax.Precision.HIGHEST)
    hl_ref[...] = lax.dot(acc[F:], m, precision=lax.Precision.HIGHEST)


_tc_finish = pl.pallas_call(
    _tc_body,
    out_shape=(
        jax.ShapeDtypeStruct((F, B), jnp.float32),
        jax.ShapeDtypeStruct((F, B), jnp.float32),
    ),
)


@jax.jit
def kernel(X, gradient, hessian):
    xt = _tc_tr(X.T)
    partials = _sc_hist(xt, gradient, hessian)
    gl, hl = _tc_finish(partials.reshape(2 * NW, 2 * F, B))
    return (gl[None], hl[None])


# speedup vs baseline: 1.7060x; 1.1371x over previous
import jax
import jax.numpy as jnp
import numpy as np
from jax import lax
from jax.experimental import pallas as pl
from jax.experimental.pallas import tpu as pltpu
from jax.experimental.pallas import tpu_sc as plsc

N = 262144
F = 64
B = 256
NC = 2
NS = 16
NW = NC * NS
SAMP = N // NW
CHUNK = 128
NCHUNK = SAMP // CHUNK
NFG = F // 16
GSZ = 16 * B
NHB = 4 * NFG
ROW_UNROLL = 8
BS = 2048


def _tc_tr_body(x_ref, o_ref):
    fi = lax.broadcasted_iota(jnp.int32, (F, F), 0)
    ci = lax.broadcasted_iota(jnp.int32, (F, F), 1)
    ident = (fi == ci).astype(jnp.bfloat16)
    x = x_ref[...].astype(jnp.bfloat16)
    xt = lax.dot_general(
        x, ident, (((0,), (0,)), ((), ())),
        precision=lax.Precision.DEFAULT,
        preferred_element_type=jnp.float32,
    )
    o_ref[...] = xt.astype(jnp.int32)


_tc_tr = pl.pallas_call(
    _tc_tr_body,
    grid=(N // BS,),
    in_specs=[pl.BlockSpec((F, BS), lambda i: (0, i))],
    out_specs=pl.BlockSpec((BS, F), lambda i: (i, 0)),
    out_shape=jax.ShapeDtypeStruct((N, F), jnp.int32),
    compiler_params=pltpu.CompilerParams(fuse_transposed_lhs_in_matmul=True),
)


def _sc_body(x_hbm, g_hbm, h_hbm, out_hbm, x_buf, g_v, h_v, *rest):
    hbufs = rest[:NHB]
    sems = rest[NHB:]
    c = lax.axis_index("c")
    s = lax.axis_index("s")
    wid = s * NC + c
    base = wid * SAMP

    def start_x(ci, b):
        pltpu.make_async_copy(
            x_hbm.at[pl.ds(base + ci * CHUNK, CHUNK)], x_buf.at[b], sems[b]
        ).start()

    def wait_x(b):
        pltpu.make_async_copy(
            x_hbm.at[pl.ds(base, CHUNK)], x_buf.at[b], sems[b]
        ).wait()

    start_x(0, 0)
    start_x(1, 1)
    pltpu.sync_copy(g_hbm.at[pl.ds(base, SAMP)], g_v)
    pltpu.sync_copy(h_hbm.at[pl.ds(base, SAMP)], h_v)

    zeros = jnp.zeros((16,), jnp.float32)

    def zero_body(i, carry):
        for hb in hbufs:
            hb[pl.ds(i * 16, 16)] = zeros
        return carry

    lax.fori_loop(0, GSZ // 16, zero_body, 0)

    lane_off = lax.iota(jnp.int32, 16) * B

    def compute_chunk(ci, b):
        def rows_body(r8, carry):
            rows = []
            for u in range(ROW_UNROLL):
                r = r8 * ROW_UNROLL + u
                gi = ci * CHUNK + r
                gidx = jnp.full((16,), gi, jnp.int32)
                gs = plsc.load_gather(g_v, [gidx])
                hs = plsc.load_gather(h_v, [gidx])
                idxs = [
                    x_buf[b, r, pl.ds(fg * 16, 16)] + lane_off
                    for fg in range(NFG)
                ]
                rows.append((gs, hs, idxs))
            for u, (gs, hs, idxs) in enumerate(rows):
                par = (u % 2) * 2 * NFG
                for fg in range(NFG):
                    plsc.addupdate_scatter(hbufs[par + fg], [idxs[fg]], gs)
                    plsc.addupdate_scatter(hbufs[par + NFG + fg], [idxs[fg]], hs)
            return carry

        lax.fori_loop(0, CHUNK // ROW_UNROLL, rows_body, 0)

    def step_body(si, carry):
        for b in range(2):
            ci = si * 2 + b
            wait_x(b)
            compute_chunk(ci, b)

            @pl.when(ci + 2 < NCHUNK)
            def _():
                start_x(ci + 2, b)

        return carry

    lax.fori_loop(0, NCHUNK // 2, step_body, 0)

    for k, hb in enumerate(hbufs):
        pltpu.sync_copy(hb, out_hbm.at[wid, k])


_sc_hist = pl.kernel(
    _sc_body,
    out_type=jax.ShapeDtypeStruct((NW, NHB, GSZ), jnp.float32),
    mesh=plsc.VectorSubcoreMesh(
        core_axis_name="c", subcore_axis_name="s", num_cores=NC, num_subcores=NS
    ),
    compiler_params=pltpu.CompilerParams(needs_layout_passes=False),
    scratch_types=[
        pltpu.VMEM((2, CHUNK, F), jnp.int32),
        pltpu.VMEM((SAMP,), jnp.float32),
        pltpu.VMEM((SAMP,), jnp.float32),
    ]
    + [pltpu.VMEM((GSZ,), jnp.float32) for _ in range(NHB)]
    + [
        pltpu.SemaphoreType.DMA,
        pltpu.SemaphoreType.DMA,
    ],
)


def _tc_body(p_ref, gl_ref, hl_ref):
    acc = jnp.sum(p_ref[...], axis=0)
    bi = lax.broadcasted_iota(jnp.int32, (B, B), 0)
    ki = lax.broadcasted_iota(jnp.int32, (B, B), 1)
    m = (bi > ki).astype(jnp.float32)
    gl_ref[...] = lax.dot(acc[:F], m, precision=lax.Precision.HIGHEST)
    hl_ref[...] = lax.dot(acc[F:], m, precision=lax.Precision.HIGHEST)


_tc_finish = pl.pallas_call(
    _tc_body,
    out_shape=(
        jax.ShapeDtypeStruct((F, B), jnp.float32),
        jax.ShapeDtypeStruct((F, B), jnp.float32),
    ),
)


@jax.jit
def kernel(X, gradient, hessian):
    partials = _sc_hist(X, gradient, hessian)
    gl, hl = _tc_finish(partials.reshape(2 * NW, 2 * F, B))
    return (gl[None], hl[None])
